# baseline (speedup 1.0000x reference)
import functools

import jax
import jax.numpy as jnp
from jax import lax
from jax.experimental import pallas as pl
from jax.experimental.pallas import tpu as pltpu
from jax.experimental.pallas import tpu_sc as plsc

_INFO = plsc.get_sparse_core_info()
_NC = _INFO.num_cores
_NS = _INFO.num_subcores
_NW = _NC * _NS


def _sc_gather(table, idx_flat, *, chunk):
    B = idx_flat.shape[0]
    D = table.shape[1]
    b_per_w = B // _NW
    n_chunks = b_per_w // chunk
    mesh = plsc.VectorSubcoreMesh(core_axis_name="c", subcore_axis_name="s")

    @functools.partial(
        pl.kernel,
        out_type=jax.ShapeDtypeStruct((B, D), jnp.float32),
        mesh=mesh,
        scratch_types=[
            pltpu.VMEM((chunk,), jnp.int32),
            pltpu.VMEM((chunk, D), jnp.float32),
            pltpu.SemaphoreType.DMA,
        ],
        compiler_params=pltpu.CompilerParams(use_tc_tiling_on_sc=False),
    )
    def k(table_hbm, idx_hbm, out_hbm, idx_v, rows_v, sem):
        wid = lax.axis_index("s") * _NC + lax.axis_index("c")
        base = wid * b_per_w

        @pl.loop(0, n_chunks)
        def _(j):
            off = base + j * chunk
            pltpu.sync_copy(idx_hbm.at[pl.ds(off, chunk)], idx_v)
            pltpu.async_copy(table_hbm.at[idx_v], rows_v, sem).wait()
            pltpu.sync_copy(rows_v, out_hbm.at[pl.ds(off, chunk)])

    return k(table, idx_flat)


def kernel(token_ids, embedding):
    orig_shape = token_ids.shape
    idx_flat = token_ids.reshape(-1).astype(jnp.int32)
    out = _sc_gather(embedding, idx_flat, chunk=1024)
    return out.reshape(*orig_shape, embedding.shape[1])


# speedup vs baseline: 1.0197x; 1.0197x over previous
import functools

import jax
import jax.numpy as jnp
from jax import lax
from jax.experimental import pallas as pl
from jax.experimental.pallas import tpu as pltpu
from jax.experimental.pallas import tpu_sc as plsc

_INFO = plsc.get_sparse_core_info()
_NC = _INFO.num_cores
_NS = _INFO.num_subcores
_NW = _NC * _NS


def _sc_gather(table, idx_flat, *, chunk):
    B = idx_flat.shape[0]
    D = table.shape[1]
    b_per_w = B // _NW
    n = b_per_w // chunk
    assert n % 2 == 0 and n >= 4
    mesh = plsc.VectorSubcoreMesh(core_axis_name="c", subcore_axis_name="s")

    @functools.partial(
        pl.kernel,
        out_type=jax.ShapeDtypeStruct((B, D), jnp.float32),
        mesh=mesh,
        scratch_types=[
            pltpu.VMEM((chunk,), jnp.int32),
            pltpu.VMEM((chunk,), jnp.int32),
            pltpu.VMEM((chunk, D), jnp.float32),
            pltpu.VMEM((chunk, D), jnp.float32),
            pltpu.SemaphoreType.DMA,
            pltpu.SemaphoreType.DMA,
            pltpu.SemaphoreType.DMA,
            pltpu.SemaphoreType.DMA,
            pltpu.SemaphoreType.DMA,
            pltpu.SemaphoreType.DMA,
        ],
        compiler_params=pltpu.CompilerParams(use_tc_tiling_on_sc=False),
    )
    def k(table_hbm, idx_hbm, out_hbm, idx0, idx1, rows0, rows1,
          g0, g1, o0, o1, i0, i1):
        wid = lax.axis_index("s") * _NC + lax.axis_index("c")
        base = wid * b_per_w
        idx_v = (idx0, idx1)
        rows_v = (rows0, rows1)
        g = (g0, g1)
        o = (o0, o1)
        i = (i0, i1)

        def idx_slice(t):
            return idx_hbm.at[pl.ds(base + t * chunk, chunk)]

        def out_slice(t):
            return out_hbm.at[pl.ds(base + t * chunk, chunk)]

        def wait_out(b):
            pltpu.make_async_copy(rows_v[b], out_slice(0), o[b]).wait()

        def step(t, b, *, wait_o, guard_prefetch):
            nb = 1 - b
            if wait_o:
                wait_out(b)
            pltpu.make_async_copy(idx_slice(t), idx_v[b], i[b]).wait()
            pltpu.async_copy(table_hbm.at[idx_v[b]], rows_v[b], g[b])
            pltpu.make_async_copy(
                table_hbm.at[idx_v[nb]], rows_v[nb], g[nb]).wait()
            pltpu.async_copy(rows_v[nb], out_slice(t - 1), o[nb])
            def prefetch():
                pltpu.async_copy(idx_slice(t + 1), idx_v[nb], i[nb])
                return None
            if guard_prefetch:
                pl.when(t + 1 < n)(prefetch)
            else:
                prefetch()

        pltpu.sync_copy(idx_slice(0), idx0)
        pltpu.async_copy(table_hbm.at[idx0], rows0, g0)
        pltpu.async_copy(idx_slice(1), idx1, i1)
        step(1, 1, wait_o=False, guard_prefetch=False)

        @pl.loop(1, n // 2)
        def _(h):
            step(2 * h, 0, wait_o=True, guard_prefetch=False)
            step(2 * h + 1, 1, wait_o=True, guard_prefetch=True)

        pltpu.make_async_copy(table_hbm.at[idx_v[1]], rows_v[1], g[1]).wait()
        pltpu.async_copy(rows_v[1], out_slice(n - 1), o[1])
        wait_out(0)
        wait_out(1)

    return k(table, idx_flat)


def kernel(token_ids, embedding):
    orig_shape = token_ids.shape
    idx_flat = token_ids.reshape(-1).astype(jnp.int32)
    out = _sc_gather(embedding, idx_flat, chunk=800)
    return out.reshape(*orig_shape, embedding.shape[1])
